# baseline (speedup 1.0000x reference)
import functools

import jax
import jax.numpy as jnp
from jax import lax
from jax.experimental import pallas as pl
from jax.experimental.pallas import tpu as pltpu
from jax.experimental.pallas import tpu_sc as plsc

T = 2048
D = 768
E = 8
DFF = 2048
TM = 128
NPAD = 2 * T + E * TM
NT = NPAD // TM
NTPAD = 48

NSUB = 16
LANES = 16


def _gating_body(x_ref, wcat_ref, noise_ref, bn_ref, ti_ref, tg_ref, hr_ref):
    x = x_ref[...]
    lg = jnp.dot(x, wcat_ref[...], preferred_element_type=jnp.float32)
    clean = lg[:, :E]
    raw = lg[:, E:] + bn_ref[...]
    std = jnp.maximum(raw, 0.0) + jnp.log1p(jnp.exp(-jnp.abs(raw))) + 1e-2
    logits = clean + noise_ref[...] * std
    iota = lax.broadcasted_iota(jnp.int32, (T, E), 1)
    m1 = jnp.max(logits, axis=1, keepdims=True)
    i1 = jnp.min(jnp.where(logits == m1, iota, E), axis=1, keepdims=True)
    masked = jnp.where(iota == i1, -jnp.inf, logits)
    m2 = jnp.max(masked, axis=1, keepdims=True)
    i2 = jnp.min(jnp.where(masked == m2, iota, E), axis=1, keepdims=True)
    s = jnp.exp(m2 - m1)
    g1 = 1.0 / (1.0 + s)
    g2 = s / (1.0 + s)
    ti_ref[...] = jnp.concatenate([i1, i2], axis=1)
    tg_ref[...] = jnp.concatenate([g1, g2], axis=1)
    iota16 = lax.broadcasted_iota(jnp.int32, (T, 2 * E), 1)
    oh = (jnp.where(iota16 == i1, 1, 0) + jnp.where(iota16 == i2, 1, 0))
    hr_ref[...] = jnp.sum(oh.reshape(NSUB, T // NSUB, 2 * E), axis=1)


def _gating(x, wcat, noise, b_noise):
    return pl.pallas_call(
        _gating_body,
        out_shape=(
            jax.ShapeDtypeStruct((T, 2), jnp.int32),
            jax.ShapeDtypeStruct((T, 2), jnp.float32),
            jax.ShapeDtypeStruct((NSUB, 2 * E), jnp.int32),
        ),
    )(x, wcat, noise, b_noise)


def _routing_body(eflat_hbm, x_hbm, hrows_hbm, xs_hbm, pos_hbm, tmap_hbm,
                  ev, hall, posv, pidx, xrows, tmapv, sem):
    cid = lax.axis_index("c")
    sid = lax.axis_index("s")
    iota = lax.iota(jnp.int32, LANES)
    napc = 2 * T // NSUB

    pltpu.sync_copy(eflat_hbm.at[pl.ds(sid * napc, napc)], ev)
    pltpu.sync_copy(hrows_hbm, hall)

    rows = [hall[pl.ds(w * LANES, LANES)] for w in range(NSUB)]
    cvec = rows[0]
    for w in range(1, NSUB):
        cvec = cvec + rows[w]
    prevsum = jnp.zeros((LANES,), jnp.int32)
    for w in range(NSUB):
        prevsum = prevsum + jnp.where(w < sid, rows[w], 0)
    pad = cvec + (TM - 1)
    pad = pad - lax.rem(pad, TM)
    eoff = plsc.cumsum(pad) - pad
    mystart = eoff + prevsum
    r = [jnp.sum(jnp.where(iota == e, mystart, 0)) for e in range(E)]

    @pl.when(jnp.logical_and(cid == 0, sid == 0))
    def _():
        et = [jnp.sum(jnp.where(iota == e, eoff, 0)) // TM for e in range(E)]
        for i in range(NTPAD // LANES):
            ivec = iota + LANES * i
            tv = jnp.full((LANES,), -1, jnp.int32)
            for e in range(E):
                tv = tv + jnp.where(ivec >= et[e], 1, 0)
            tmapv[pl.ds(LANES * i, LANES)] = tv
        pltpu.sync_copy(tmapv, tmap_hbm)

    for j in range(napc // LANES):
        v = ev[pl.ds(LANES * j, LANES)]
        pv = jnp.zeros((LANES,), jnp.int32)
        for e in range(E):
            m = v == e
            ones = jnp.where(m, 1, 0)
            cs = plsc.cumsum(ones)
            pv = jnp.where(m, r[e] + cs - 1, pv)
            r[e] = r[e] + jnp.sum(ones)
        posv[pl.ds(LANES * j, LANES)] = pv

    @pl.when(cid == 0)
    def _():
        pltpu.sync_copy(posv, pos_hbm.at[pl.ds(sid * napc, napc)])

    tpc = T // NSUB
    for j in range(tpc // LANES):
        idxv = 2 * (LANES * j + iota) + cid
        pidx[pl.ds(LANES * j, LANES)] = plsc.load_gather(posv, [idxv])

    pltpu.sync_copy(x_hbm.at[pl.ds(sid * tpc, tpc)], xrows)
    pltpu.async_copy(xrows, xs_hbm.at[pidx], sem).wait()


def _routing(eflat, x, hrows):
    mesh = plsc.VectorSubcoreMesh(core_axis_name="c", subcore_axis_name="s",
                                  num_cores=2, num_subcores=NSUB)
    return pl.kernel(
        _routing_body,
        out_type=(
            jax.ShapeDtypeStruct((NPAD, D), jnp.float32),
            jax.ShapeDtypeStruct((2 * T,), jnp.int32),
            jax.ShapeDtypeStruct((NTPAD,), jnp.int32),
        ),
        mesh=mesh,
        scratch_types=(
            pltpu.VMEM((2 * T // NSUB,), jnp.int32),
            pltpu.VMEM((NSUB * LANES,), jnp.int32),
            pltpu.VMEM((2 * T // NSUB,), jnp.int32),
            pltpu.VMEM((T // NSUB,), jnp.int32),
            pltpu.VMEM((T // NSUB, D), jnp.float32),
            pltpu.VMEM((NTPAD,), jnp.int32),
            pltpu.SemaphoreType.DMA,
        ),
        compiler_params=pltpu.CompilerParams(needs_layout_passes=False),
    )(eflat, x, hrows)


def _ffn_body(tm_ref, xs_ref, We1_ref, be1_ref, We2_ref, be2_ref, ys_ref):
    xb = xs_ref[...].astype(jnp.bfloat16)
    h = jnp.dot(xb, We1_ref[0], preferred_element_type=jnp.float32)
    h = jnp.maximum(h + be1_ref[0], 0.0).astype(jnp.bfloat16)
    y = jnp.dot(h, We2_ref[0], preferred_element_type=jnp.float32)
    ys_ref[...] = y + be2_ref[0]


def _ffn(tmap, xs, We1, be1, We2, be2):
    grid_spec = pltpu.PrefetchScalarGridSpec(
        num_scalar_prefetch=1,
        grid=(NT,),
        in_specs=[
            pl.BlockSpec((TM, D), lambda i, tm: (i, 0)),
            pl.BlockSpec((1, D, DFF), lambda i, tm: (tm[i], 0, 0)),
            pl.BlockSpec((1, 1, DFF), lambda i, tm: (tm[i], 0, 0)),
            pl.BlockSpec((1, DFF, D), lambda i, tm: (tm[i], 0, 0)),
            pl.BlockSpec((1, 1, D), lambda i, tm: (tm[i], 0, 0)),
        ],
        out_specs=pl.BlockSpec((TM, D), lambda i, tm: (i, 0)),
    )
    return pl.pallas_call(
        _ffn_body,
        grid_spec=grid_spec,
        out_shape=jax.ShapeDtypeStruct((NPAD, D), jnp.float32),
        compiler_params=pltpu.CompilerParams(
            dimension_semantics=("arbitrary",),
        ),
    )(tmap, xs, We1.astype(jnp.bfloat16), be1.reshape(E, 1, DFF),
      We2.astype(jnp.bfloat16), be2.reshape(E, 1, D))


def _combine_body(ys_hbm, pos_hbm, g_hbm, out_hbm,
                  posv, s0, s1, gv, rowsA, rowsB, sem):
    cid = lax.axis_index("c")
    sid = lax.axis_index("s")
    iota = lax.iota(jnp.int32, LANES)
    NW = 2 * NSUB
    tpw = T // NW
    wid = sid * 2 + cid

    pltpu.sync_copy(pos_hbm.at[pl.ds(wid * 2 * tpw, 2 * tpw)], posv)
    pltpu.sync_copy(g_hbm.at[pl.ds(wid * 2 * tpw, 2 * tpw)], gv)
    for j in range(tpw // LANES):
        idxv = 2 * (LANES * j + iota)
        s0[pl.ds(LANES * j, LANES)] = plsc.load_gather(posv, [idxv])
        s1[pl.ds(LANES * j, LANES)] = plsc.load_gather(posv, [idxv + 1])
    d0 = pltpu.async_copy(ys_hbm.at[s0], rowsA, sem)
    d0.wait()
    d1 = pltpu.async_copy(ys_hbm.at[s1], rowsB, sem)
    d1.wait()

    def body(j, carry):
        g0 = plsc.load_gather(gv, [jnp.zeros((LANES,), jnp.int32) + 2 * j])
        g1 = plsc.load_gather(gv, [jnp.zeros((LANES,), jnp.int32) + 2 * j + 1])
        for kk in range(D // LANES):
            a = rowsA[j, pl.ds(LANES * kk, LANES)]
            b = rowsB[j, pl.ds(LANES * kk, LANES)]
            rowsA[j, pl.ds(LANES * kk, LANES)] = a * g0 + b * g1
        return carry

    lax.fori_loop(0, tpw, body, jnp.int32(0))
    pltpu.sync_copy(rowsA, out_hbm.at[pl.ds(wid * tpw, tpw)])


def _combine(ys, posflat, gflat):
    mesh = plsc.VectorSubcoreMesh(core_axis_name="c", subcore_axis_name="s",
                                  num_cores=2, num_subcores=NSUB)
    NW = 2 * NSUB
    tpw = T // NW
    return pl.kernel(
        _combine_body,
        out_type=jax.ShapeDtypeStruct((T, D), jnp.float32),
        mesh=mesh,
        scratch_types=(
            pltpu.VMEM((2 * tpw,), jnp.int32),
            pltpu.VMEM((tpw,), jnp.int32),
            pltpu.VMEM((tpw,), jnp.int32),
            pltpu.VMEM((2 * tpw,), jnp.float32),
            pltpu.VMEM((tpw, D), jnp.float32),
            pltpu.VMEM((tpw, D), jnp.float32),
            pltpu.SemaphoreType.DMA,
        ),
        compiler_params=pltpu.CompilerParams(needs_layout_passes=False),
    )(ys, posflat, gflat)


def kernel(x, noise, w_gate, w_noise, b_noise, We1, be1, We2, be2):
    wcat = jnp.concatenate([w_gate, w_noise], axis=1)
    top_idx, top_gates, hrows = _gating(x, wcat, noise, b_noise)
    eflat = top_idx.reshape((2 * T,))
    xs, posflat, tmap = _routing(eflat, x, hrows.reshape((NSUB * LANES,)))
    ys = _ffn(tmap, xs, We1, be1, We2, be2)
    out = _combine(ys, posflat, top_gates.reshape((2 * T,)))
    return out


# speedup vs baseline: 1.2137x; 1.2137x over previous
import functools

import jax
import jax.numpy as jnp
from jax import lax
from jax.experimental import pallas as pl
from jax.experimental.pallas import tpu as pltpu
from jax.experimental.pallas import tpu_sc as plsc

T = 2048
D = 768
E = 8
DFF = 2048
TM = 128
NPAD = 2 * T + E * TM
NT = NPAD // TM
NTPAD = 48

NSUB = 16
LANES = 16


def _gating_body(x_ref, wcat_ref, noise_ref, bn_ref, ti_ref, tg_ref, hr_ref):
    x = x_ref[...]
    lg = jnp.dot(x, wcat_ref[...], preferred_element_type=jnp.float32)
    clean = lg[:, :E]
    raw = lg[:, E:] + bn_ref[...]
    std = jnp.maximum(raw, 0.0) + jnp.log1p(jnp.exp(-jnp.abs(raw))) + 1e-2
    logits = clean + noise_ref[...] * std
    iota = lax.broadcasted_iota(jnp.int32, (T, E), 1)
    m1 = jnp.max(logits, axis=1, keepdims=True)
    i1 = jnp.min(jnp.where(logits == m1, iota, E), axis=1, keepdims=True)
    masked = jnp.where(iota == i1, -jnp.inf, logits)
    m2 = jnp.max(masked, axis=1, keepdims=True)
    i2 = jnp.min(jnp.where(masked == m2, iota, E), axis=1, keepdims=True)
    s = jnp.exp(m2 - m1)
    g1 = 1.0 / (1.0 + s)
    g2 = s / (1.0 + s)
    ti_ref[...] = jnp.concatenate([i1, i2], axis=1)
    tg_ref[...] = jnp.concatenate([g1, g2], axis=1)
    iota16 = lax.broadcasted_iota(jnp.int32, (T, 2 * E), 1)
    oh = (jnp.where(iota16 == i1, 1, 0) + jnp.where(iota16 == i2, 1, 0))
    hr_ref[...] = jnp.sum(oh.reshape(NSUB, T // NSUB, 2 * E), axis=1)


def _gating(x, wcat, noise, b_noise):
    return pl.pallas_call(
        _gating_body,
        out_shape=(
            jax.ShapeDtypeStruct((T, 2), jnp.int32),
            jax.ShapeDtypeStruct((T, 2), jnp.float32),
            jax.ShapeDtypeStruct((NSUB, 2 * E), jnp.int32),
        ),
    )(x, wcat, noise, b_noise)


def _routing_body(eflat_hbm, x_hbm, hrows_hbm, xs_hbm, pos_hbm, tmap_hbm,
                  ev, hall, posv, pidx, xrows, tmapv, sem):
    cid = lax.axis_index("c")
    sid = lax.axis_index("s")
    iota = lax.iota(jnp.int32, LANES)
    napc = 2 * T // NSUB

    pltpu.sync_copy(eflat_hbm.at[pl.ds(sid * napc, napc)], ev)
    pltpu.sync_copy(hrows_hbm, hall)

    rows = [hall[pl.ds(w * LANES, LANES)] for w in range(NSUB)]
    cvec = rows[0]
    for w in range(1, NSUB):
        cvec = cvec + rows[w]
    prevsum = jnp.zeros((LANES,), jnp.int32)
    for w in range(NSUB):
        prevsum = prevsum + jnp.where(w < sid, rows[w], 0)
    pad = cvec + (TM - 1)
    pad = pad - lax.rem(pad, TM)
    eoff = plsc.cumsum(pad) - pad
    mystart = eoff + prevsum
    r = [jnp.sum(jnp.where(iota == e, mystart, 0)) for e in range(E)]

    @pl.when(jnp.logical_and(cid == 0, sid == 0))
    def _():
        et = [jnp.sum(jnp.where(iota == e, eoff, 0)) // TM for e in range(E)]
        for i in range(NTPAD // LANES):
            ivec = iota + LANES * i
            tv = jnp.full((LANES,), -1, jnp.int32)
            for e in range(E):
                tv = tv + jnp.where(ivec >= et[e], 1, 0)
            tmapv[pl.ds(LANES * i, LANES)] = tv
        pltpu.sync_copy(tmapv, tmap_hbm)

    for j in range(napc // LANES):
        v = ev[pl.ds(LANES * j, LANES)]
        pv = jnp.zeros((LANES,), jnp.int32)
        for e in range(E):
            m = v == e
            ones = jnp.where(m, 1, 0)
            cs = plsc.cumsum(ones)
            pv = jnp.where(m, r[e] + cs - 1, pv)
            r[e] = r[e] + jnp.sum(ones)
        posv[pl.ds(LANES * j, LANES)] = pv

    @pl.when(cid == 0)
    def _():
        pltpu.sync_copy(posv, pos_hbm.at[pl.ds(sid * napc, napc)])

    tpc = T // NSUB
    for j in range(tpc // LANES):
        idxv = 2 * (LANES * j + iota) + cid
        pidx[pl.ds(LANES * j, LANES)] = plsc.load_gather(posv, [idxv])

    pltpu.sync_copy(x_hbm.at[pl.ds(sid * tpc, tpc)], xrows)
    pltpu.async_copy(xrows, xs_hbm.at[pidx], sem).wait()


def _routing(eflat, x, hrows):
    mesh = plsc.VectorSubcoreMesh(core_axis_name="c", subcore_axis_name="s",
                                  num_cores=2, num_subcores=NSUB)
    return pl.kernel(
        _routing_body,
        out_type=(
            jax.ShapeDtypeStruct((NPAD, D), jnp.float32),
            jax.ShapeDtypeStruct((2 * T,), jnp.int32),
            jax.ShapeDtypeStruct((NTPAD,), jnp.int32),
        ),
        mesh=mesh,
        scratch_types=(
            pltpu.VMEM((2 * T // NSUB,), jnp.int32),
            pltpu.VMEM((NSUB * LANES,), jnp.int32),
            pltpu.VMEM((2 * T // NSUB,), jnp.int32),
            pltpu.VMEM((T // NSUB,), jnp.int32),
            pltpu.VMEM((T // NSUB, D), jnp.float32),
            pltpu.VMEM((NTPAD,), jnp.int32),
            pltpu.SemaphoreType.DMA,
        ),
        compiler_params=pltpu.CompilerParams(needs_layout_passes=False),
    )(eflat, x, hrows)


def _ffn_body(tm_ref, xs_ref, We1_ref, be1_ref, We2_ref, be2_ref, ys_ref):
    h = jnp.dot(xs_ref[...], We1_ref[0], preferred_element_type=jnp.float32)
    h = jnp.maximum(h + be1_ref[0], 0.0)
    y = jnp.dot(h, We2_ref[0], preferred_element_type=jnp.float32)
    ys_ref[...] = y + be2_ref[0]


def _ffn(tmap, xs, We1, be1, We2, be2):
    grid_spec = pltpu.PrefetchScalarGridSpec(
        num_scalar_prefetch=1,
        grid=(NT,),
        in_specs=[
            pl.BlockSpec((TM, D), lambda i, tm: (i, 0)),
            pl.BlockSpec((1, D, DFF), lambda i, tm: (tm[i], 0, 0)),
            pl.BlockSpec((1, 1, DFF), lambda i, tm: (tm[i], 0, 0)),
            pl.BlockSpec((1, DFF, D), lambda i, tm: (tm[i], 0, 0)),
            pl.BlockSpec((1, 1, D), lambda i, tm: (tm[i], 0, 0)),
        ],
        out_specs=pl.BlockSpec((TM, D), lambda i, tm: (i, 0)),
    )
    return pl.pallas_call(
        _ffn_body,
        grid_spec=grid_spec,
        out_shape=jax.ShapeDtypeStruct((NPAD, D), jnp.float32),
        compiler_params=pltpu.CompilerParams(
            dimension_semantics=("arbitrary",),
        ),
    )(tmap, xs, We1, be1.reshape(E, 1, DFF), We2, be2.reshape(E, 1, D))


def _combine_body(ys_hbm, pos_hbm, g_hbm, out_hbm,
                  posv, s0, s1, gv, rowsA, rowsB, sem):
    cid = lax.axis_index("c")
    sid = lax.axis_index("s")
    iota = lax.iota(jnp.int32, LANES)
    NW = 2 * NSUB
    tpw = T // NW
    wid = sid * 2 + cid

    pltpu.sync_copy(pos_hbm.at[pl.ds(wid * 2 * tpw, 2 * tpw)], posv)
    pltpu.sync_copy(g_hbm.at[pl.ds(wid * 2 * tpw, 2 * tpw)], gv)
    for j in range(tpw // LANES):
        idxv = 2 * (LANES * j + iota)
        s0[pl.ds(LANES * j, LANES)] = plsc.load_gather(posv, [idxv])
        s1[pl.ds(LANES * j, LANES)] = plsc.load_gather(posv, [idxv + 1])
    d0 = pltpu.async_copy(ys_hbm.at[s0], rowsA, sem)
    d0.wait()
    d1 = pltpu.async_copy(ys_hbm.at[s1], rowsB, sem)
    d1.wait()

    def body(j, carry):
        g0 = plsc.load_gather(gv, [jnp.zeros((LANES,), jnp.int32) + 2 * j])
        g1 = plsc.load_gather(gv, [jnp.zeros((LANES,), jnp.int32) + 2 * j + 1])
        for kk in range(D // LANES):
            a = rowsA[j, pl.ds(LANES * kk, LANES)]
            b = rowsB[j, pl.ds(LANES * kk, LANES)]
            rowsA[j, pl.ds(LANES * kk, LANES)] = a * g0 + b * g1
        return carry

    lax.fori_loop(0, tpw, body, jnp.int32(0))
    pltpu.sync_copy(rowsA, out_hbm.at[pl.ds(wid * tpw, tpw)])


def _combine(ys, posflat, gflat):
    mesh = plsc.VectorSubcoreMesh(core_axis_name="c", subcore_axis_name="s",
                                  num_cores=2, num_subcores=NSUB)
    NW = 2 * NSUB
    tpw = T // NW
    return pl.kernel(
        _combine_body,
        out_type=jax.ShapeDtypeStruct((T, D), jnp.float32),
        mesh=mesh,
        scratch_types=(
            pltpu.VMEM((2 * tpw,), jnp.int32),
            pltpu.VMEM((tpw,), jnp.int32),
            pltpu.VMEM((tpw,), jnp.int32),
            pltpu.VMEM((2 * tpw,), jnp.float32),
            pltpu.VMEM((tpw, D), jnp.float32),
            pltpu.VMEM((tpw, D), jnp.float32),
            pltpu.SemaphoreType.DMA,
        ),
        compiler_params=pltpu.CompilerParams(needs_layout_passes=False),
    )(ys, posflat, gflat)


def kernel(x, noise, w_gate, w_noise, b_noise, We1, be1, We2, be2):
    wcat = jnp.concatenate([w_gate, w_noise], axis=1)
    top_idx, top_gates, hrows = _gating(x, wcat, noise, b_noise)
    eflat = top_idx.reshape((2 * T,))
    xs, posflat, tmap = _routing(eflat, x, hrows.reshape((NSUB * LANES,)))
    ys = _ffn(tmap, xs, We1, be1, We2, be2)
    out = _combine(ys, posflat, top_gates.reshape((2 * T,)))
    return out


# speedup vs baseline: 1.2415x; 1.0229x over previous
import functools

import jax
import jax.numpy as jnp
from jax import lax
from jax.experimental import pallas as pl
from jax.experimental.pallas import tpu as pltpu
from jax.experimental.pallas import tpu_sc as plsc

T = 2048
D = 768
E = 8
DFF = 2048
TM = 128
NPAD = 2 * T + E * TM
NT = NPAD // TM
NTPAD = 48

NSUB = 16
LANES = 16


def _gating_body(x_ref, wcat_ref, noise_ref, bn_ref, ti_ref, tg_ref, hr_ref):
    x = x_ref[...]
    lg = jnp.dot(x, wcat_ref[...], preferred_element_type=jnp.float32)
    clean = lg[:, :E]
    raw = lg[:, E:] + bn_ref[...]
    std = jnp.maximum(raw, 0.0) + jnp.log1p(jnp.exp(-jnp.abs(raw))) + 1e-2
    logits = clean + noise_ref[...] * std
    iota = lax.broadcasted_iota(jnp.int32, (T, E), 1)
    m1 = jnp.max(logits, axis=1, keepdims=True)
    i1 = jnp.min(jnp.where(logits == m1, iota, E), axis=1, keepdims=True)
    masked = jnp.where(iota == i1, -jnp.inf, logits)
    m2 = jnp.max(masked, axis=1, keepdims=True)
    i2 = jnp.min(jnp.where(masked == m2, iota, E), axis=1, keepdims=True)
    s = jnp.exp(m2 - m1)
    g1 = 1.0 / (1.0 + s)
    g2 = s / (1.0 + s)
    ti_ref[...] = jnp.concatenate([i1, i2], axis=1)
    tg_ref[...] = jnp.concatenate([g1, g2], axis=1)
    iota16 = lax.broadcasted_iota(jnp.int32, (T, 2 * E), 1)
    oh = (jnp.where(iota16 == i1, 1, 0) + jnp.where(iota16 == i2, 1, 0))
    hr_ref[...] = jnp.sum(oh.reshape(NSUB, T // NSUB, 2 * E), axis=1)


def _gating(x, wcat, noise, b_noise):
    return pl.pallas_call(
        _gating_body,
        out_shape=(
            jax.ShapeDtypeStruct((T, 2), jnp.int32),
            jax.ShapeDtypeStruct((T, 2), jnp.float32),
            jax.ShapeDtypeStruct((NSUB, 2 * E), jnp.int32),
        ),
    )(x, wcat, noise, b_noise)


def _routing_body(eflat_hbm, x_hbm, hrows_hbm, xs_hbm, pos_hbm, tmap_hbm,
                  ev, hall, posv, pidx, xrows, tmapv, sem):
    cid = lax.axis_index("c")
    sid = lax.axis_index("s")
    iota = lax.iota(jnp.int32, LANES)
    napc = 2 * T // NSUB

    pltpu.sync_copy(eflat_hbm.at[pl.ds(sid * napc, napc)], ev)
    pltpu.sync_copy(hrows_hbm, hall)

    rows = [hall[pl.ds(w * LANES, LANES)] for w in range(NSUB)]
    cvec = rows[0]
    for w in range(1, NSUB):
        cvec = cvec + rows[w]
    prevsum = jnp.zeros((LANES,), jnp.int32)
    for w in range(NSUB):
        prevsum = prevsum + jnp.where(w < sid, rows[w], 0)
    pad = cvec + (TM - 1)
    pad = pad - lax.rem(pad, TM)
    eoff = plsc.cumsum(pad) - pad
    mystart = eoff + prevsum
    r = [jnp.sum(jnp.where(iota == e, mystart, 0)) for e in range(E)]

    @pl.when(jnp.logical_and(cid == 0, sid == 0))
    def _():
        et = [jnp.sum(jnp.where(iota == e, eoff, 0)) // TM for e in range(E)]
        ntiles = jnp.sum(pad) // TM
        for i in range(NTPAD // LANES):
            ivec = iota + LANES * i
            tv = jnp.full((LANES,), -1, jnp.int32)
            for e in range(E):
                tv = tv + jnp.where(ivec >= et[e], 1, 0)
            if i == NTPAD // LANES - 1:
                tv = jnp.where(iota == LANES - 1, ntiles, tv)
            tmapv[pl.ds(LANES * i, LANES)] = tv
        pltpu.sync_copy(tmapv, tmap_hbm)

    for j in range(napc // LANES):
        v = ev[pl.ds(LANES * j, LANES)]
        pv = jnp.zeros((LANES,), jnp.int32)
        for e in range(E):
            m = v == e
            ones = jnp.where(m, 1, 0)
            cs = plsc.cumsum(ones)
            pv = jnp.where(m, r[e] + cs - 1, pv)
            r[e] = r[e] + jnp.sum(ones)
        posv[pl.ds(LANES * j, LANES)] = pv

    @pl.when(cid == 0)
    def _():
        pltpu.sync_copy(posv, pos_hbm.at[pl.ds(sid * napc, napc)])

    tpc = T // NSUB
    for j in range(tpc // LANES):
        idxv = 2 * (LANES * j + iota) + cid
        pidx[pl.ds(LANES * j, LANES)] = plsc.load_gather(posv, [idxv])

    pltpu.sync_copy(x_hbm.at[pl.ds(sid * tpc, tpc)], xrows)
    pltpu.async_copy(xrows, xs_hbm.at[pidx], sem).wait()


def _routing(eflat, x, hrows):
    mesh = plsc.VectorSubcoreMesh(core_axis_name="c", subcore_axis_name="s",
                                  num_cores=2, num_subcores=NSUB)
    return pl.kernel(
        _routing_body,
        out_type=(
            jax.ShapeDtypeStruct((NPAD, D), jnp.float32),
            jax.ShapeDtypeStruct((2 * T,), jnp.int32),
            jax.ShapeDtypeStruct((NTPAD,), jnp.int32),
        ),
        mesh=mesh,
        scratch_types=(
            pltpu.VMEM((2 * T // NSUB,), jnp.int32),
            pltpu.VMEM((NSUB * LANES,), jnp.int32),
            pltpu.VMEM((2 * T // NSUB,), jnp.int32),
            pltpu.VMEM((T // NSUB,), jnp.int32),
            pltpu.VMEM((T // NSUB, D), jnp.float32),
            pltpu.VMEM((NTPAD,), jnp.int32),
            pltpu.SemaphoreType.DMA,
        ),
        compiler_params=pltpu.CompilerParams(needs_layout_passes=False),
    )(eflat, x, hrows)


def _ffn_body(tm_ref, xs_ref, We1_ref, be1_ref, We2_ref, be2_ref, ys_ref):
    @pl.when(pl.program_id(0) < tm_ref[NTPAD - 1])
    def _():
        h = jnp.dot(xs_ref[...], We1_ref[0], preferred_element_type=jnp.float32)
        h = jnp.maximum(h + be1_ref[0], 0.0)
        y = jnp.dot(h, We2_ref[0], preferred_element_type=jnp.float32)
        ys_ref[...] = y + be2_ref[0]


def _ffn(tmap, xs, We1, be1, We2, be2):
    grid_spec = pltpu.PrefetchScalarGridSpec(
        num_scalar_prefetch=1,
        grid=(NT,),
        in_specs=[
            pl.BlockSpec((TM, D), lambda i, tm: (i, 0)),
            pl.BlockSpec((1, D, DFF), lambda i, tm: (tm[i], 0, 0)),
            pl.BlockSpec((1, 1, DFF), lambda i, tm: (tm[i], 0, 0)),
            pl.BlockSpec((1, DFF, D), lambda i, tm: (tm[i], 0, 0)),
            pl.BlockSpec((1, 1, D), lambda i, tm: (tm[i], 0, 0)),
        ],
        out_specs=pl.BlockSpec((TM, D), lambda i, tm: (i, 0)),
    )
    return pl.pallas_call(
        _ffn_body,
        grid_spec=grid_spec,
        out_shape=jax.ShapeDtypeStruct((NPAD, D), jnp.float32),
        compiler_params=pltpu.CompilerParams(
            dimension_semantics=("arbitrary",),
        ),
    )(tmap, xs, We1, be1.reshape(E, 1, DFF), We2, be2.reshape(E, 1, D))


def _combine_body(ys_hbm, pos_hbm, g_hbm, out_hbm,
                  posv, s0, s1, gv, rowsA, rowsB, sem):
    cid = lax.axis_index("c")
    sid = lax.axis_index("s")
    iota = lax.iota(jnp.int32, LANES)
    NW = 2 * NSUB
    tpw = T // NW
    wid = sid * 2 + cid

    pltpu.sync_copy(pos_hbm.at[pl.ds(wid * 2 * tpw, 2 * tpw)], posv)
    pltpu.sync_copy(g_hbm.at[pl.ds(wid * 2 * tpw, 2 * tpw)], gv)
    for j in range(tpw // LANES):
        idxv = 2 * (LANES * j + iota)
        s0[pl.ds(LANES * j, LANES)] = plsc.load_gather(posv, [idxv])
        s1[pl.ds(LANES * j, LANES)] = plsc.load_gather(posv, [idxv + 1])
    d0 = pltpu.async_copy(ys_hbm.at[s0], rowsA, sem)
    d0.wait()
    d1 = pltpu.async_copy(ys_hbm.at[s1], rowsB, sem)
    d1.wait()

    def body(j, carry):
        g0 = plsc.load_gather(gv, [jnp.zeros((LANES,), jnp.int32) + 2 * j])
        g1 = plsc.load_gather(gv, [jnp.zeros((LANES,), jnp.int32) + 2 * j + 1])
        for kk in range(D // LANES):
            a = rowsA[j, pl.ds(LANES * kk, LANES)]
            b = rowsB[j, pl.ds(LANES * kk, LANES)]
            rowsA[j, pl.ds(LANES * kk, LANES)] = a * g0 + b * g1
        return carry

    lax.fori_loop(0, tpw, body, jnp.int32(0))
    pltpu.sync_copy(rowsA, out_hbm.at[pl.ds(wid * tpw, tpw)])


def _combine(ys, posflat, gflat):
    mesh = plsc.VectorSubcoreMesh(core_axis_name="c", subcore_axis_name="s",
                                  num_cores=2, num_subcores=NSUB)
    NW = 2 * NSUB
    tpw = T // NW
    return pl.kernel(
        _combine_body,
        out_type=jax.ShapeDtypeStruct((T, D), jnp.float32),
        mesh=mesh,
        scratch_types=(
            pltpu.VMEM((2 * tpw,), jnp.int32),
            pltpu.VMEM((tpw,), jnp.int32),
            pltpu.VMEM((tpw,), jnp.int32),
            pltpu.VMEM((2 * tpw,), jnp.float32),
            pltpu.VMEM((tpw, D), jnp.float32),
            pltpu.VMEM((tpw, D), jnp.float32),
            pltpu.SemaphoreType.DMA,
        ),
        compiler_params=pltpu.CompilerParams(needs_layout_passes=False),
    )(ys, posflat, gflat)


def kernel(x, noise, w_gate, w_noise, b_noise, We1, be1, We2, be2):
    wcat = jnp.concatenate([w_gate, w_noise], axis=1)
    top_idx, top_gates, hrows = _gating(x, wcat, noise, b_noise)
    eflat = top_idx.reshape((2 * T,))
    xs, posflat, tmap = _routing(eflat, x, hrows.reshape((NSUB * LANES,)))
    ys = _ffn(tmap, xs, We1, be1, We2, be2)
    out = _combine(ys, posflat, top_gates.reshape((2 * T,)))
    return out


# speedup vs baseline: 1.3315x; 1.0725x over previous
import functools

import jax
import jax.numpy as jnp
from jax import lax
from jax.experimental import pallas as pl
from jax.experimental.pallas import tpu as pltpu
from jax.experimental.pallas import tpu_sc as plsc

T = 2048
D = 768
E = 8
DFF = 2048
TM = 256
NPAD = 2 * T + E * TM
NT = NPAD // TM
NTPAD = 32

NSUB = 16
LANES = 16


def _gating_body(x_ref, wcat_ref, noise_ref, bn_ref, ti_ref, tg_ref, hr_ref):
    x = x_ref[...]
    lg = jnp.dot(x, wcat_ref[...], preferred_element_type=jnp.float32)
    clean = lg[:, :E]
    raw = lg[:, E:] + bn_ref[...]
    std = jnp.maximum(raw, 0.0) + jnp.log1p(jnp.exp(-jnp.abs(raw))) + 1e-2
    logits = clean + noise_ref[...] * std
    iota = lax.broadcasted_iota(jnp.int32, (T, E), 1)
    m1 = jnp.max(logits, axis=1, keepdims=True)
    i1 = jnp.min(jnp.where(logits == m1, iota, E), axis=1, keepdims=True)
    masked = jnp.where(iota == i1, -jnp.inf, logits)
    m2 = jnp.max(masked, axis=1, keepdims=True)
    i2 = jnp.min(jnp.where(masked == m2, iota, E), axis=1, keepdims=True)
    s = jnp.exp(m2 - m1)
    g1 = 1.0 / (1.0 + s)
    g2 = s / (1.0 + s)
    ti_ref[...] = jnp.concatenate([i1, i2], axis=1)
    tg_ref[...] = jnp.concatenate([g1, g2], axis=1)
    iota16 = lax.broadcasted_iota(jnp.int32, (T, 2 * E), 1)
    oh = (jnp.where(iota16 == i1, 1, 0) + jnp.where(iota16 == i2, 1, 0))
    hr_ref[...] = jnp.sum(oh.reshape(NSUB, T // NSUB, 2 * E), axis=1)


def _gating(x, wcat, noise, b_noise):
    return pl.pallas_call(
        _gating_body,
        out_shape=(
            jax.ShapeDtypeStruct((T, 2), jnp.int32),
            jax.ShapeDtypeStruct((T, 2), jnp.float32),
            jax.ShapeDtypeStruct((NSUB, 2 * E), jnp.int32),
        ),
    )(x, wcat, noise, b_noise)


def _routing_body(eflat_hbm, x_hbm, hrows_hbm, xs_hbm, pos_hbm, tmap_hbm,
                  ev, hall, posv, pidx, xrows, tmapv, sem):
    cid = lax.axis_index("c")
    sid = lax.axis_index("s")
    iota = lax.iota(jnp.int32, LANES)
    napc = 2 * T // NSUB

    pltpu.sync_copy(eflat_hbm.at[pl.ds(sid * napc, napc)], ev)
    pltpu.sync_copy(hrows_hbm, hall)

    rows = [hall[pl.ds(w * LANES, LANES)] for w in range(NSUB)]
    cvec = rows[0]
    for w in range(1, NSUB):
        cvec = cvec + rows[w]
    prevsum = jnp.zeros((LANES,), jnp.int32)
    for w in range(NSUB):
        prevsum = prevsum + jnp.where(w < sid, rows[w], 0)
    pad = cvec + (TM - 1)
    pad = pad - lax.rem(pad, TM)
    eoff = plsc.cumsum(pad) - pad
    mystart = eoff + prevsum
    r = [jnp.sum(jnp.where(iota == e, mystart, 0)) for e in range(E)]

    @pl.when(jnp.logical_and(cid == 0, sid == 0))
    def _():
        et = [jnp.sum(jnp.where(iota == e, eoff, 0)) // TM for e in range(E)]
        ntiles = jnp.sum(pad) // TM
        for i in range(NTPAD // LANES):
            ivec = iota + LANES * i
            tv = jnp.full((LANES,), -1, jnp.int32)
            for e in range(E):
                tv = tv + jnp.where(ivec >= et[e], 1, 0)
            if i == NTPAD // LANES - 1:
                tv = jnp.where(iota == LANES - 1, ntiles, tv)
            tmapv[pl.ds(LANES * i, LANES)] = tv
        pltpu.sync_copy(tmapv, tmap_hbm)

    for j in range(napc // LANES):
        v = ev[pl.ds(LANES * j, LANES)]
        pv = jnp.zeros((LANES,), jnp.int32)
        for e in range(E):
            m = v == e
            ones = jnp.where(m, 1, 0)
            cs = plsc.cumsum(ones)
            pv = jnp.where(m, r[e] + cs - 1, pv)
            r[e] = r[e] + jnp.sum(ones)
        posv[pl.ds(LANES * j, LANES)] = pv

    @pl.when(cid == 0)
    def _():
        pltpu.sync_copy(posv, pos_hbm.at[pl.ds(sid * napc, napc)])

    tpc = T // NSUB
    for j in range(tpc // LANES):
        idxv = 2 * (LANES * j + iota) + cid
        pidx[pl.ds(LANES * j, LANES)] = plsc.load_gather(posv, [idxv])

    pltpu.sync_copy(x_hbm.at[pl.ds(sid * tpc, tpc)], xrows)
    pltpu.async_copy(xrows, xs_hbm.at[pidx], sem).wait()


def _routing(eflat, x, hrows):
    mesh = plsc.VectorSubcoreMesh(core_axis_name="c", subcore_axis_name="s",
                                  num_cores=2, num_subcores=NSUB)
    return pl.kernel(
        _routing_body,
        out_type=(
            jax.ShapeDtypeStruct((NPAD, D), jnp.float32),
            jax.ShapeDtypeStruct((2 * T,), jnp.int32),
            jax.ShapeDtypeStruct((NTPAD,), jnp.int32),
        ),
        mesh=mesh,
        scratch_types=(
            pltpu.VMEM((2 * T // NSUB,), jnp.int32),
            pltpu.VMEM((NSUB * LANES,), jnp.int32),
            pltpu.VMEM((2 * T // NSUB,), jnp.int32),
            pltpu.VMEM((T // NSUB,), jnp.int32),
            pltpu.VMEM((T // NSUB, D), jnp.float32),
            pltpu.VMEM((NTPAD,), jnp.int32),
            pltpu.SemaphoreType.DMA,
        ),
        compiler_params=pltpu.CompilerParams(needs_layout_passes=False),
    )(eflat, x, hrows)


def _ffn_body(tm_ref, xs_ref, We1_ref, be1_ref, We2_ref, be2_ref, ys_ref):
    @pl.when(pl.program_id(0) < tm_ref[NTPAD - 1])
    def _():
        h = jnp.dot(xs_ref[...], We1_ref[0], preferred_element_type=jnp.float32)
        h = jnp.maximum(h + be1_ref[0], 0.0)
        y = jnp.dot(h, We2_ref[0], preferred_element_type=jnp.float32)
        ys_ref[...] = y + be2_ref[0]


def _ffn(tmap, xs, We1, be1, We2, be2):
    grid_spec = pltpu.PrefetchScalarGridSpec(
        num_scalar_prefetch=1,
        grid=(NT,),
        in_specs=[
            pl.BlockSpec((TM, D), lambda i, tm: (i, 0)),
            pl.BlockSpec((1, D, DFF), lambda i, tm: (tm[i], 0, 0)),
            pl.BlockSpec((1, 1, DFF), lambda i, tm: (tm[i], 0, 0)),
            pl.BlockSpec((1, DFF, D), lambda i, tm: (tm[i], 0, 0)),
            pl.BlockSpec((1, 1, D), lambda i, tm: (tm[i], 0, 0)),
        ],
        out_specs=pl.BlockSpec((TM, D), lambda i, tm: (i, 0)),
    )
    return pl.pallas_call(
        _ffn_body,
        grid_spec=grid_spec,
        out_shape=jax.ShapeDtypeStruct((NPAD, D), jnp.float32),
        compiler_params=pltpu.CompilerParams(
            dimension_semantics=("arbitrary",),
        ),
    )(tmap, xs, We1, be1.reshape(E, 1, DFF), We2, be2.reshape(E, 1, D))


def _combine_body(ys_hbm, pos_hbm, g_hbm, out_hbm,
                  posv, s0, s1, gv, rowsA, rowsB, sem):
    cid = lax.axis_index("c")
    sid = lax.axis_index("s")
    iota = lax.iota(jnp.int32, LANES)
    NW = 2 * NSUB
    tpw = T // NW
    wid = sid * 2 + cid

    pltpu.sync_copy(pos_hbm.at[pl.ds(wid * 2 * tpw, 2 * tpw)], posv)
    pltpu.sync_copy(g_hbm.at[pl.ds(wid * 2 * tpw, 2 * tpw)], gv)
    for j in range(tpw // LANES):
        idxv = 2 * (LANES * j + iota)
        s0[pl.ds(LANES * j, LANES)] = plsc.load_gather(posv, [idxv])
        s1[pl.ds(LANES * j, LANES)] = plsc.load_gather(posv, [idxv + 1])
    d0 = pltpu.async_copy(ys_hbm.at[s0], rowsA, sem)
    d0.wait()
    d1 = pltpu.async_copy(ys_hbm.at[s1], rowsB, sem)
    d1.wait()

    def body(j, carry):
        g0 = plsc.load_gather(gv, [jnp.zeros((LANES,), jnp.int32) + 2 * j])
        g1 = plsc.load_gather(gv, [jnp.zeros((LANES,), jnp.int32) + 2 * j + 1])
        for kk in range(D // LANES):
            a = rowsA[j, pl.ds(LANES * kk, LANES)]
            b = rowsB[j, pl.ds(LANES * kk, LANES)]
            rowsA[j, pl.ds(LANES * kk, LANES)] = a * g0 + b * g1
        return carry

    lax.fori_loop(0, tpw, body, jnp.int32(0))
    pltpu.sync_copy(rowsA, out_hbm.at[pl.ds(wid * tpw, tpw)])


def _combine(ys, posflat, gflat):
    mesh = plsc.VectorSubcoreMesh(core_axis_name="c", subcore_axis_name="s",
                                  num_cores=2, num_subcores=NSUB)
    NW = 2 * NSUB
    tpw = T // NW
    return pl.kernel(
        _combine_body,
        out_type=jax.ShapeDtypeStruct((T, D), jnp.float32),
        mesh=mesh,
        scratch_types=(
            pltpu.VMEM((2 * tpw,), jnp.int32),
            pltpu.VMEM((tpw,), jnp.int32),
            pltpu.VMEM((tpw,), jnp.int32),
            pltpu.VMEM((2 * tpw,), jnp.float32),
            pltpu.VMEM((tpw, D), jnp.float32),
            pltpu.VMEM((tpw, D), jnp.float32),
            pltpu.SemaphoreType.DMA,
        ),
        compiler_params=pltpu.CompilerParams(needs_layout_passes=False),
    )(ys, posflat, gflat)


def kernel(x, noise, w_gate, w_noise, b_noise, We1, be1, We2, be2):
    wcat = jnp.concatenate([w_gate, w_noise], axis=1)
    top_idx, top_gates, hrows = _gating(x, wcat, noise, b_noise)
    eflat = top_idx.reshape((2 * T,))
    xs, posflat, tmap = _routing(eflat, x, hrows.reshape((NSUB * LANES,)))
    ys = _ffn(tmap, xs, We1, be1, We2, be2)
    out = _combine(ys, posflat, top_gates.reshape((2 * T,)))
    return out
